# baseline (device time: 174743 ns/iter reference)
import jax
import jax.numpy as jnp
from jax import lax
from jax.experimental import pallas as pl
from jax.experimental.pallas import tpu as pltpu

N_DEV = 4
M_CHUNK = 1024


def kernel(x, w_mat, scale_x, scale_w):
    m_glob, k_per = x.shape
    k_per2, n = w_mat.shape
    assert k_per == k_per2 and m_glob == N_DEV * M_CHUNK

    def body(x_ref, w_ref, sx_ref, sw_ref, out_ref,
             wb_ref, comm_ref, send_sems, recv_sems):
        p = lax.axis_index("i")
        right = lax.rem(p + 1, N_DEV)
        left = lax.rem(p + N_DEV - 1, N_DEV)

        barrier_sem = pltpu.get_barrier_semaphore()
        for nbr in (left, right):
            pl.semaphore_signal(
                barrier_sem, inc=1,
                device_id=(nbr,), device_id_type=pl.DeviceIdType.MESH,
            )
        pl.semaphore_wait(barrier_sem, 2)

        wb_ref[...] = w_ref[...].astype(jnp.bfloat16)

        def gemm(c):
            xc = x_ref[pl.ds(c * M_CHUNK, M_CHUNK), :].astype(jnp.bfloat16)
            return lax.dot_general(
                xc, wb_ref[...],
                dimension_numbers=(((1,), (0,)), ((), ())),
                preferred_element_type=jnp.float32,
            )

        comm_ref[0, :, :] = gemm(lax.rem(p + N_DEV - 1, N_DEV)).astype(
            jnp.bfloat16)

        for h in range(N_DEV - 1):
            ss = h % 2
            rs = (h + 1) % 2
            rdma = pltpu.make_async_remote_copy(
                src_ref=comm_ref.at[ss],
                dst_ref=comm_ref.at[rs],
                send_sem=send_sems.at[ss],
                recv_sem=recv_sems.at[rs],
                device_id=(right,),
                device_id_type=pl.DeviceIdType.MESH,
            )
            rdma.start()
            g = gemm(lax.rem(p + (N_DEV - 2 - h), N_DEV))
            rdma.wait()
            if h < N_DEV - 2:
                comm_ref[rs, :, :] = (
                    g + comm_ref[rs, :, :].astype(jnp.float32)
                ).astype(jnp.bfloat16)
            else:
                alpha = sx_ref[0] * sw_ref[0]
                y = (g + comm_ref[rs, :, :].astype(jnp.float32)) * alpha
                out_ref[...] = jnp.maximum(y, 0.0)

    return pl.pallas_call(
        body,
        out_shape=jax.ShapeDtypeStruct((M_CHUNK, n), jnp.float32),
        in_specs=[
            pl.BlockSpec(memory_space=pltpu.VMEM),
            pl.BlockSpec(memory_space=pltpu.VMEM),
            pl.BlockSpec(memory_space=pltpu.SMEM),
            pl.BlockSpec(memory_space=pltpu.SMEM),
        ],
        out_specs=pl.BlockSpec(memory_space=pltpu.VMEM),
        scratch_shapes=[
            pltpu.VMEM((k_per, n), jnp.bfloat16),
            pltpu.VMEM((2, M_CHUNK, n), jnp.bfloat16),
            pltpu.SemaphoreType.DMA((2,)),
            pltpu.SemaphoreType.DMA((2,)),
        ],
        compiler_params=pltpu.CompilerParams(collective_id=0),
    )(x, w_mat, scale_x, scale_w)


# device time: 107364 ns/iter; 1.6276x vs baseline; 1.6276x over previous
import jax
import jax.numpy as jnp
from jax import lax
from jax.experimental import pallas as pl
from jax.experimental.pallas import tpu as pltpu

N_DEV = 4
M_CHUNK = 1024


def kernel(x, w_mat, scale_x, scale_w):
    m_glob, k_per = x.shape
    k_per2, n = w_mat.shape
    assert k_per == k_per2 and m_glob == N_DEV * M_CHUNK
    nh = n // 2

    def body(x_ref, w_ref, sx_ref, sw_ref, out_ref,
             wb_ref, cw_ref, ccw_ref,
             send_cw, recv_cw, send_ccw, recv_ccw):
        p = lax.axis_index("i")
        right = lax.rem(p + 1, N_DEV)
        left = lax.rem(p + N_DEV - 1, N_DEV)

        barrier_sem = pltpu.get_barrier_semaphore()
        for nbr in (left, right):
            pl.semaphore_signal(
                barrier_sem, inc=1,
                device_id=(nbr,), device_id_type=pl.DeviceIdType.MESH,
            )
        pl.semaphore_wait(barrier_sem, 2)

        wb_ref[...] = w_ref[...].astype(jnp.bfloat16)

        def gemm(c, half):
            xc = x_ref[pl.ds(c * M_CHUNK, M_CHUNK), :].astype(jnp.bfloat16)
            return lax.dot_general(
                xc, wb_ref[:, half * nh:(half + 1) * nh],
                dimension_numbers=(((1,), (0,)), ((), ())),
                preferred_element_type=jnp.float32,
            )

        cw_ref[0, :, :] = gemm(lax.rem(p + N_DEV - 1, N_DEV), 0).astype(
            jnp.bfloat16)
        ccw_ref[0, :, :] = gemm(lax.rem(p + 1, N_DEV), 1).astype(
            jnp.bfloat16)

        for h in range(N_DEV - 1):
            ss = h % 2
            rs = (h + 1) % 2
            rdma_cw = pltpu.make_async_remote_copy(
                src_ref=cw_ref.at[ss],
                dst_ref=cw_ref.at[rs],
                send_sem=send_cw.at[ss],
                recv_sem=recv_cw.at[rs],
                device_id=(right,),
                device_id_type=pl.DeviceIdType.MESH,
            )
            rdma_ccw = pltpu.make_async_remote_copy(
                src_ref=ccw_ref.at[ss],
                dst_ref=ccw_ref.at[rs],
                send_sem=send_ccw.at[ss],
                recv_sem=recv_ccw.at[rs],
                device_id=(left,),
                device_id_type=pl.DeviceIdType.MESH,
            )
            rdma_cw.start()
            rdma_ccw.start()
            g_cw = gemm(lax.rem(p + (N_DEV - 2 - h), N_DEV), 0)
            g_ccw = gemm(lax.rem(p + h + 2, N_DEV), 1)
            rdma_cw.wait()
            rdma_ccw.wait()
            if h < N_DEV - 2:
                cw_ref[rs, :, :] = (
                    g_cw + cw_ref[rs, :, :].astype(jnp.float32)
                ).astype(jnp.bfloat16)
                ccw_ref[rs, :, :] = (
                    g_ccw + ccw_ref[rs, :, :].astype(jnp.float32)
                ).astype(jnp.bfloat16)
            else:
                alpha = sx_ref[0] * sw_ref[0]
                y_cw = (g_cw + cw_ref[rs, :, :].astype(jnp.float32)) * alpha
                y_ccw = (g_ccw + ccw_ref[rs, :, :].astype(jnp.float32)) * alpha
                out_ref[:, :nh] = jnp.maximum(y_cw, 0.0)
                out_ref[:, nh:] = jnp.maximum(y_ccw, 0.0)

    return pl.pallas_call(
        body,
        out_shape=jax.ShapeDtypeStruct((M_CHUNK, n), jnp.float32),
        in_specs=[
            pl.BlockSpec(memory_space=pltpu.VMEM),
            pl.BlockSpec(memory_space=pltpu.VMEM),
            pl.BlockSpec(memory_space=pltpu.SMEM),
            pl.BlockSpec(memory_space=pltpu.SMEM),
        ],
        out_specs=pl.BlockSpec(memory_space=pltpu.VMEM),
        scratch_shapes=[
            pltpu.VMEM((k_per, n), jnp.bfloat16),
            pltpu.VMEM((2, M_CHUNK, nh), jnp.bfloat16),
            pltpu.VMEM((2, M_CHUNK, nh), jnp.bfloat16),
            pltpu.SemaphoreType.DMA((2,)),
            pltpu.SemaphoreType.DMA((2,)),
            pltpu.SemaphoreType.DMA((2,)),
            pltpu.SemaphoreType.DMA((2,)),
        ],
        compiler_params=pltpu.CompilerParams(collective_id=0),
    )(x, w_mat, scale_x, scale_w)


# device time: 93355 ns/iter; 1.8718x vs baseline; 1.1501x over previous
import jax
import jax.numpy as jnp
from jax import lax
from jax.experimental import pallas as pl
from jax.experimental.pallas import tpu as pltpu

N_DEV = 4
M_CHUNK = 1024
N_SUB = 2


def kernel(x, w_mat, scale_x, scale_w):
    m_glob, k_per = x.shape
    k_per2, n = w_mat.shape
    assert k_per == k_per2 and m_glob == N_DEV * M_CHUNK
    nh = n // 2
    ns = nh // N_SUB

    f8 = jnp.float8_e5m2

    def body(x_ref, w_ref, sx_ref, sw_ref, out_ref,
             wb_ref, cw_ref, ccw_ref,
             cw_send, cw_recv, ccw_send, ccw_recv):
        p = lax.axis_index("i")
        right = lax.rem(p + 1, N_DEV)
        left = lax.rem(p + N_DEV - 1, N_DEV)

        barrier_sem = pltpu.get_barrier_semaphore()
        for nbr in (left, right):
            pl.semaphore_signal(
                barrier_sem, inc=1,
                device_id=(nbr,), device_id_type=pl.DeviceIdType.MESH,
            )
        wb_ref[...] = w_ref[...].astype(f8)
        pl.semaphore_wait(barrier_sem, 2)

        def gemm(c, lo):
            xc = x_ref[pl.ds(c * M_CHUNK, M_CHUNK), :].astype(f8)
            return lax.dot_general(
                xc, wb_ref[:, lo:lo + nh],
                dimension_numbers=(((1,), (0,)), ((), ())),
                preferred_element_type=jnp.float32,
            )

        def mk(buf, sems_s, sems_r, h, sub, dev):
            sl = pl.ds(sub * ns, ns)
            return pltpu.make_async_remote_copy(
                src_ref=buf.at[h, :, sl],
                dst_ref=buf.at[h + 1, :, sl],
                send_sem=sems_s.at[h, sub],
                recv_sem=sems_r.at[h, sub],
                device_id=(dev,),
                device_id_type=pl.DeviceIdType.MESH,
            )

        def mk_cw(h, sub):
            return mk(cw_ref, cw_send, cw_recv, h, sub, right)

        def mk_ccw(h, sub):
            return mk(ccw_ref, ccw_send, ccw_recv, h, sub, left)

        cw_ref[0, :, :] = gemm(lax.rem(p + N_DEV - 1, N_DEV), 0).astype(
            jnp.bfloat16)
        cw_h = [[mk_cw(h, s) for s in range(N_SUB)] for h in range(3)]
        ccw_h = [[mk_ccw(h, s) for s in range(N_SUB)] for h in range(3)]
        for s in range(N_SUB):
            cw_h[0][s].start()
        ccw_ref[0, :, :] = gemm(lax.rem(p + 1, N_DEV), nh).astype(
            jnp.bfloat16)
        for s in range(N_SUB):
            ccw_h[0][s].start()

        g_cw = gemm(lax.rem(p + 2, N_DEV), 0)
        g_ccw = gemm(lax.rem(p + 2, N_DEV), nh)

        for h in range(2):
            for s in range(N_SUB):
                sl = pl.ds(s * ns, ns)
                cw_h[h][s].wait_recv()
                cw_ref[h + 1, :, sl] = (
                    g_cw[:, s * ns:(s + 1) * ns]
                    + cw_ref[h + 1, :, sl].astype(jnp.float32)
                ).astype(jnp.bfloat16)
                if h > 0:
                    cw_h[h - 1][s].wait_send()
                cw_h[h + 1][s].start()
                ccw_h[h][s].wait_recv()
                ccw_ref[h + 1, :, sl] = (
                    g_ccw[:, s * ns:(s + 1) * ns]
                    + ccw_ref[h + 1, :, sl].astype(jnp.float32)
                ).astype(jnp.bfloat16)
                if h > 0:
                    ccw_h[h - 1][s].wait_send()
                ccw_h[h + 1][s].start()
            if h == 0:
                g_cw = gemm(lax.rem(p + 1, N_DEV), 0)
                g_ccw = gemm(lax.rem(p + 3, N_DEV), nh)

        g_cw = gemm(p, 0)
        g_ccw = gemm(p, nh)
        alpha = sx_ref[0] * sw_ref[0]

        for s in range(N_SUB):
            sl = pl.ds(s * ns, ns)
            cw_h[2][s].wait_recv()
            y = (g_cw[:, s * ns:(s + 1) * ns]
                 + cw_ref[3, :, sl].astype(jnp.float32)) * alpha
            out_ref[:, sl] = jnp.maximum(y, 0.0)
            ccw_h[2][s].wait_recv()
            y = (g_ccw[:, s * ns:(s + 1) * ns]
                 + ccw_ref[3, :, sl].astype(jnp.float32)) * alpha
            out_ref[:, pl.ds(nh + s * ns, ns)] = jnp.maximum(y, 0.0)

        for s in range(N_SUB):
            cw_h[1][s].wait_send()
            ccw_h[1][s].wait_send()
            cw_h[2][s].wait_send()
            ccw_h[2][s].wait_send()

    return pl.pallas_call(
        body,
        out_shape=jax.ShapeDtypeStruct((M_CHUNK, n), jnp.float32),
        in_specs=[
            pl.BlockSpec(memory_space=pltpu.VMEM),
            pl.BlockSpec(memory_space=pltpu.VMEM),
            pl.BlockSpec(memory_space=pltpu.SMEM),
            pl.BlockSpec(memory_space=pltpu.SMEM),
        ],
        out_specs=pl.BlockSpec(memory_space=pltpu.VMEM),
        scratch_shapes=[
            pltpu.VMEM((k_per, n), f8),
            pltpu.VMEM((4, M_CHUNK, nh), jnp.bfloat16),
            pltpu.VMEM((4, M_CHUNK, nh), jnp.bfloat16),
            pltpu.SemaphoreType.DMA((3, N_SUB)),
            pltpu.SemaphoreType.DMA((3, N_SUB)),
            pltpu.SemaphoreType.DMA((3, N_SUB)),
            pltpu.SemaphoreType.DMA((3, N_SUB)),
        ],
        compiler_params=pltpu.CompilerParams(
            collective_id=0, vmem_limit_bytes=100 * 1024 * 1024,
        ),
    )(x, w_mat, scale_x, scale_w)


# device time: 93089 ns/iter; 1.8772x vs baseline; 1.0029x over previous
import jax
import jax.numpy as jnp
from jax import lax
from jax.experimental import pallas as pl
from jax.experimental.pallas import tpu as pltpu

N_DEV = 4
M_CHUNK = 1024
N_SUB = 2


def kernel(x, w_mat, scale_x, scale_w):
    m_glob, k_per = x.shape
    k_per2, n = w_mat.shape
    assert k_per == k_per2 and m_glob == N_DEV * M_CHUNK
    nh = n // 2
    ns = nh // N_SUB

    f8 = jnp.float8_e5m2

    def body(x_ref, w_ref, sx_ref, sw_ref, out_ref,
             wb_ref, cw_ref, ccw_ref,
             cw_send, cw_recv, ccw_send, ccw_recv):
        p = lax.axis_index("i")
        right = lax.rem(p + 1, N_DEV)
        left = lax.rem(p + N_DEV - 1, N_DEV)

        barrier_sem = pltpu.get_barrier_semaphore()
        for nbr in (left, right):
            pl.semaphore_signal(
                barrier_sem, inc=1,
                device_id=(nbr,), device_id_type=pl.DeviceIdType.MESH,
            )
        wb_ref[...] = w_ref[...].astype(f8)
        pl.semaphore_wait(barrier_sem, 2)

        def gemm(c, lo):
            xc = x_ref[pl.ds(c * M_CHUNK, M_CHUNK), :].astype(f8)
            return lax.dot_general(
                xc, wb_ref[:, lo:lo + ns],
                dimension_numbers=(((1,), (0,)), ((), ())),
                preferred_element_type=jnp.float32,
            )

        def mk(buf, sems_s, sems_r, h, sub, dev):
            sl = pl.ds(sub * ns, ns)
            return pltpu.make_async_remote_copy(
                src_ref=buf.at[h, :, sl],
                dst_ref=buf.at[h + 1, :, sl],
                send_sem=sems_s.at[h, sub],
                recv_sem=sems_r.at[h, sub],
                device_id=(dev,),
                device_id_type=pl.DeviceIdType.MESH,
            )

        def mk_cw(h, sub):
            return mk(cw_ref, cw_send, cw_recv, h, sub, right)

        def mk_ccw(h, sub):
            return mk(ccw_ref, ccw_send, ccw_recv, h, sub, left)

        cw_h = [[mk_cw(h, s) for s in range(N_SUB)] for h in range(3)]
        ccw_h = [[mk_ccw(h, s) for s in range(N_SUB)] for h in range(3)]

        c_cw = [lax.rem(p + k, N_DEV) for k in (3, 2, 1, 0)]
        c_ccw = [lax.rem(p + k, N_DEV) for k in (1, 2, 3, 0)]

        for s in range(N_SUB):
            sl = pl.ds(s * ns, ns)
            cw_ref[0, :, sl] = gemm(c_cw[0], s * ns).astype(jnp.bfloat16)
            cw_h[0][s].start()
            ccw_ref[0, :, sl] = gemm(c_ccw[0], nh + s * ns).astype(
                jnp.bfloat16)
            ccw_h[0][s].start()

        for h in range(2):
            for s in range(N_SUB):
                sl = pl.ds(s * ns, ns)
                g = gemm(c_cw[h + 1], s * ns)
                cw_h[h][s].wait_recv()
                cw_ref[h + 1, :, sl] = (
                    g + cw_ref[h + 1, :, sl].astype(jnp.float32)
                ).astype(jnp.bfloat16)
                if h > 0:
                    cw_h[h - 1][s].wait_send()
                cw_h[h + 1][s].start()
                g = gemm(c_ccw[h + 1], nh + s * ns)
                ccw_h[h][s].wait_recv()
                ccw_ref[h + 1, :, sl] = (
                    g + ccw_ref[h + 1, :, sl].astype(jnp.float32)
                ).astype(jnp.bfloat16)
                if h > 0:
                    ccw_h[h - 1][s].wait_send()
                ccw_h[h + 1][s].start()

        alpha = sx_ref[0] * sw_ref[0]

        for s in range(N_SUB):
            sl = pl.ds(s * ns, ns)
            g = gemm(c_cw[3], s * ns)
            cw_h[2][s].wait_recv()
            y = (g + cw_ref[3, :, sl].astype(jnp.float32)) * alpha
            out_ref[:, sl] = jnp.maximum(y, 0.0)
            g = gemm(c_ccw[3], nh + s * ns)
            ccw_h[2][s].wait_recv()
            y = (g + ccw_ref[3, :, sl].astype(jnp.float32)) * alpha
            out_ref[:, pl.ds(nh + s * ns, ns)] = jnp.maximum(y, 0.0)

        for s in range(N_SUB):
            cw_h[1][s].wait_send()
            ccw_h[1][s].wait_send()
            cw_h[2][s].wait_send()
            ccw_h[2][s].wait_send()

    return pl.pallas_call(
        body,
        out_shape=jax.ShapeDtypeStruct((M_CHUNK, n), jnp.float32),
        in_specs=[
            pl.BlockSpec(memory_space=pltpu.VMEM),
            pl.BlockSpec(memory_space=pltpu.VMEM),
            pl.BlockSpec(memory_space=pltpu.SMEM),
            pl.BlockSpec(memory_space=pltpu.SMEM),
        ],
        out_specs=pl.BlockSpec(memory_space=pltpu.VMEM),
        scratch_shapes=[
            pltpu.VMEM((k_per, n), f8),
            pltpu.VMEM((4, M_CHUNK, nh), jnp.bfloat16),
            pltpu.VMEM((4, M_CHUNK, nh), jnp.bfloat16),
            pltpu.SemaphoreType.DMA((3, N_SUB)),
            pltpu.SemaphoreType.DMA((3, N_SUB)),
            pltpu.SemaphoreType.DMA((3, N_SUB)),
            pltpu.SemaphoreType.DMA((3, N_SUB)),
        ],
        compiler_params=pltpu.CompilerParams(
            collective_id=0, vmem_limit_bytes=100 * 1024 * 1024,
        ),
    )(x, w_mat, scale_x, scale_w)


# device time: 92062 ns/iter; 1.8981x vs baseline; 1.0112x over previous
import jax
import jax.numpy as jnp
from jax import lax
from jax.experimental import pallas as pl
from jax.experimental.pallas import tpu as pltpu

N_DEV = 4
M_CHUNK = 1024
N_SUB = 4


def kernel(x, w_mat, scale_x, scale_w):
    m_glob, k_per = x.shape
    k_per2, n = w_mat.shape
    assert k_per == k_per2 and m_glob == N_DEV * M_CHUNK
    nh = n // 2
    ns = nh // N_SUB

    f8 = jnp.float8_e5m2

    def body(x_ref, w_ref, sx_ref, sw_ref, out_ref,
             wb_ref, cw_ref, ccw_ref,
             cw_send, cw_recv, ccw_send, ccw_recv):
        p = lax.axis_index("i")
        right = lax.rem(p + 1, N_DEV)
        left = lax.rem(p + N_DEV - 1, N_DEV)

        barrier_sem = pltpu.get_barrier_semaphore()
        for nbr in (left, right):
            pl.semaphore_signal(
                barrier_sem, inc=1,
                device_id=(nbr,), device_id_type=pl.DeviceIdType.MESH,
            )
        wb_ref[...] = w_ref[...].astype(f8)
        pl.semaphore_wait(barrier_sem, 2)

        def gemm(c, lo):
            xc = x_ref[pl.ds(c * M_CHUNK, M_CHUNK), :].astype(f8)
            return lax.dot_general(
                xc, wb_ref[:, lo:lo + ns],
                dimension_numbers=(((1,), (0,)), ((), ())),
                preferred_element_type=jnp.float32,
            )

        def mk(buf, sems_s, sems_r, h, sub, dev):
            sl = pl.ds(sub * ns, ns)
            return pltpu.make_async_remote_copy(
                src_ref=buf.at[h, :, sl],
                dst_ref=buf.at[h + 1, :, sl],
                send_sem=sems_s.at[h, sub],
                recv_sem=sems_r.at[h, sub],
                device_id=(dev,),
                device_id_type=pl.DeviceIdType.MESH,
            )

        def mk_cw(h, sub):
            return mk(cw_ref, cw_send, cw_recv, h, sub, right)

        def mk_ccw(h, sub):
            return mk(ccw_ref, ccw_send, ccw_recv, h, sub, left)

        cw_h = [[mk_cw(h, s) for s in range(N_SUB)] for h in range(3)]
        ccw_h = [[mk_ccw(h, s) for s in range(N_SUB)] for h in range(3)]

        c_cw = [lax.rem(p + k, N_DEV) for k in (3, 2, 1, 0)]
        c_ccw = [lax.rem(p + k, N_DEV) for k in (1, 2, 3, 0)]

        for s in range(N_SUB):
            sl = pl.ds(s * ns, ns)
            cw_ref[0, :, sl] = gemm(c_cw[0], s * ns).astype(jnp.bfloat16)
            cw_h[0][s].start()
            ccw_ref[0, :, sl] = gemm(c_ccw[0], nh + s * ns).astype(
                jnp.bfloat16)
            ccw_h[0][s].start()

        for h in range(2):
            for s in range(N_SUB):
                sl = pl.ds(s * ns, ns)
                g = gemm(c_cw[h + 1], s * ns)
                cw_h[h][s].wait_recv()
                cw_ref[h + 1, :, sl] = (
                    g + cw_ref[h + 1, :, sl].astype(jnp.float32)
                ).astype(jnp.bfloat16)
                if h > 0:
                    cw_h[h - 1][s].wait_send()
                cw_h[h + 1][s].start()
                g = gemm(c_ccw[h + 1], nh + s * ns)
                ccw_h[h][s].wait_recv()
                ccw_ref[h + 1, :, sl] = (
                    g + ccw_ref[h + 1, :, sl].astype(jnp.float32)
                ).astype(jnp.bfloat16)
                if h > 0:
                    ccw_h[h - 1][s].wait_send()
                ccw_h[h + 1][s].start()

        alpha = sx_ref[0] * sw_ref[0]

        for s in range(N_SUB):
            sl = pl.ds(s * ns, ns)
            g = gemm(c_cw[3], s * ns)
            cw_h[2][s].wait_recv()
            y = (g + cw_ref[3, :, sl].astype(jnp.float32)) * alpha
            out_ref[:, sl] = jnp.maximum(y, 0.0)
            g = gemm(c_ccw[3], nh + s * ns)
            ccw_h[2][s].wait_recv()
            y = (g + ccw_ref[3, :, sl].astype(jnp.float32)) * alpha
            out_ref[:, pl.ds(nh + s * ns, ns)] = jnp.maximum(y, 0.0)

        for s in range(N_SUB):
            cw_h[1][s].wait_send()
            ccw_h[1][s].wait_send()
            cw_h[2][s].wait_send()
            ccw_h[2][s].wait_send()

    return pl.pallas_call(
        body,
        out_shape=jax.ShapeDtypeStruct((M_CHUNK, n), jnp.float32),
        in_specs=[
            pl.BlockSpec(memory_space=pltpu.VMEM),
            pl.BlockSpec(memory_space=pltpu.VMEM),
            pl.BlockSpec(memory_space=pltpu.SMEM),
            pl.BlockSpec(memory_space=pltpu.SMEM),
        ],
        out_specs=pl.BlockSpec(memory_space=pltpu.VMEM),
        scratch_shapes=[
            pltpu.VMEM((k_per, n), f8),
            pltpu.VMEM((4, M_CHUNK, nh), jnp.bfloat16),
            pltpu.VMEM((4, M_CHUNK, nh), jnp.bfloat16),
            pltpu.SemaphoreType.DMA((3, N_SUB)),
            pltpu.SemaphoreType.DMA((3, N_SUB)),
            pltpu.SemaphoreType.DMA((3, N_SUB)),
            pltpu.SemaphoreType.DMA((3, N_SUB)),
        ],
        compiler_params=pltpu.CompilerParams(
            collective_id=0, vmem_limit_bytes=100 * 1024 * 1024,
        ),
    )(x, w_mat, scale_x, scale_w)
